# baseline (speedup 1.0000x reference)
import jax
import jax.numpy as jnp
from jax import lax
from jax.experimental import pallas as pl
from jax.experimental.pallas import tpu as pltpu
from jax.experimental.pallas import tpu_sc as plsc

N = 100000
E = 6400000
L = 50
NCORES = 2
NSUB = 16
NW = NCORES * NSUB
TS = 6272
HTS = TS // 2
NPAD = NSUB * TS
NROWS = NPAD // 128
CHUNK = 2048
CR = CHUNK // 128
EROWS = E // 128
NCHUNKS = E // CHUNK
BASE_CH = NCHUNKS // NW
EXTRA_CH = NCHUNKS % NW


def _sc_body(comb_hbm, src_hbm, dst_hbm, w_hbm, b_hbm,
             part0_hbm, part1_hbm,
             comb_v, w_v, b_v, src_a, src_b, dst_a, dst_b, msg_a, msg_b,
             idx_a, idx_b, ri_a, ri_b, stage_ri, node_v, out_sh,
             sem_in, sem_sc):
    srcs = (src_a, src_b)
    dsts = (dst_a, dst_b)
    msgs = (msg_a, msg_b)
    idxs = (idx_a, idx_b)
    ris = (ri_a, ri_b)
    core = lax.axis_index("c")
    sid = lax.axis_index("s")
    wid = core * NSUB + sid
    lanes = lax.iota(jnp.int32, 16)

    def _sri(j, _):
        stage_ri[pl.ds(j * 16, 16)] = j * 16 + lanes
        return _
    lax.fori_loop(0, NROWS // 16, _sri, None)
    pltpu.sync_copy(comb_hbm.at[stage_ri], comb_v)
    pltpu.sync_copy(w_hbm, w_v)
    pltpu.sync_copy(b_hbm, b_v)

    def _zero(j, _):
        node_v[pl.ds(j * 16, 16)] = jnp.zeros((16,), jnp.float32)
        return _
    lax.fori_loop(0, HTS // 16, _zero, None)
    for h in range(2):
        pltpu.sync_copy(node_v, out_sh.at[pl.ds(sid * TS + h * HTS, HTS)])
    plsc.subcore_barrier()

    nch = BASE_CH + jnp.where(wid < EXTRA_CH, 1, 0)

    ris[0][...] = wid * CR + lanes
    pltpu.sync_copy(src_hbm.at[ris[0]], srcs[0])
    pltpu.sync_copy(dst_hbm.at[ris[0]], dsts[0])

    def _one_chunk(k, p):
        row0 = jnp.minimum((wid + (k + 1) * NW), NCHUNKS - 1) * CR
        ris[1 - p][...] = row0 + lanes
        pltpu.async_copy(src_hbm.at[ris[1 - p]], srcs[1 - p], sem_in)
        pltpu.async_copy(dst_hbm.at[ris[1 - p]], dsts[1 - p], sem_in)

        def _vec(j, _2):
            for u in range(4):
                o = j * 64 + u * 16
                r = o // 128
                cpos = o % 128
                s = srcs[p][r, pl.ds(cpos, 16)]
                d = dsts[p][r, pl.ds(cpos, 16)]
                ws = plsc.load_gather(comb_v, [s >> 7, s & 127])
                wd = plsc.load_gather(comb_v, [d >> 7, d & 127])
                lab_s = ws & 0xFF
                lab_d = wd & 0xFF
                w = plsc.load_gather(w_v, [lab_d * L + lab_s])
                xf = plsc.bitcast(ws & jnp.int32(-65536), jnp.float32)
                msgs[p][pl.ds(o, 16)] = w * xf
                idxs[p][pl.ds(o, 16)] = d
            return _2
        lax.fori_loop(0, CHUNK // 64, _vec, None)

        @pl.when(k > 0)
        def _drain_prev():
            pltpu.make_async_copy(
                msgs[1 - p], out_sh.at[idxs[1 - p]], sem_sc).wait()

        pltpu.async_copy(msgs[p], out_sh.at[idxs[p]], sem_sc, add=True)
        pltpu.make_async_copy(src_hbm.at[ris[1 - p]], srcs[1 - p], sem_in).wait()
        pltpu.make_async_copy(dst_hbm.at[ris[1 - p]], dsts[1 - p], sem_in).wait()

    def _pair(i, _):
        for half in range(2):
            k = 2 * i + half

            @pl.when(k < nch)
            def _do(k=k, half=half):
                _one_chunk(k, half)
        return _
    lax.fori_loop(0, (BASE_CH + 2) // 2, _pair, None)

    @pl.when(nch % 2 == 1)
    def _drain_last0():
        pltpu.make_async_copy(msgs[0], out_sh.at[idxs[0]], sem_sc).wait()

    @pl.when(nch % 2 == 0)
    def _drain_last1():
        pltpu.make_async_copy(msgs[1], out_sh.at[idxs[1]], sem_sc).wait()

    plsc.subcore_barrier()

    for h in range(2):
        base = sid * TS + h * HTS
        pltpu.sync_copy(out_sh.at[pl.ds(base, HTS)], node_v)

        @pl.when(core == 0)
        def _bias(base=base):
            def _b(j, _):
                o = base + j * 16
                word = comb_v[o // 128, pl.ds(o % 128, 16)]
                bias = plsc.load_gather(b_v, [word & 0xFF])
                node_v[pl.ds(j * 16, 16)] = node_v[pl.ds(j * 16, 16)] + bias
                return _
            lax.fori_loop(0, HTS // 16, _b, None)

        @pl.when(core == 0)
        def _wb0(base=base):
            pltpu.sync_copy(node_v, part0_hbm.at[pl.ds(base, HTS)])

        @pl.when(core == 1)
        def _wb1(base=base):
            pltpu.sync_copy(node_v, part1_hbm.at[pl.ds(base, HTS)])


_sc_call = pl.kernel(
    _sc_body,
    out_type=(jax.ShapeDtypeStruct((NPAD,), jnp.float32),
              jax.ShapeDtypeStruct((NPAD,), jnp.float32)),
    mesh=plsc.VectorSubcoreMesh(core_axis_name="c", subcore_axis_name="s"),
    compiler_params=pltpu.CompilerParams(needs_layout_passes=False),
    scratch_types=[
        pltpu.VMEM((NROWS, 128), jnp.int32),
        pltpu.VMEM((L * L,), jnp.float32),
        pltpu.VMEM((L,), jnp.float32),
        pltpu.VMEM((CR, 128), jnp.int32),
        pltpu.VMEM((CR, 128), jnp.int32),
        pltpu.VMEM((CR, 128), jnp.int32),
        pltpu.VMEM((CR, 128), jnp.int32),
        pltpu.VMEM((CHUNK,), jnp.float32),
        pltpu.VMEM((CHUNK,), jnp.float32),
        pltpu.VMEM((CHUNK,), jnp.int32),
        pltpu.VMEM((CHUNK,), jnp.int32),
        pltpu.VMEM((16,), jnp.int32),
        pltpu.VMEM((16,), jnp.int32),
        pltpu.VMEM((NROWS,), jnp.int32),
        pltpu.VMEM((HTS,), jnp.float32),
        pltpu.VMEM_SHARED((NPAD,), jnp.float32),
        pltpu.SemaphoreType.DMA,
        pltpu.SemaphoreType.DMA,
    ],
)


def _combine_body(a_ref, b_ref, o_ref):
    o_ref[...] = a_ref[...] + b_ref[...]


def kernel(x, edge_index, node_labels, edge_props, Param_W, Param_b):
    del edge_props
    src = edge_index[0].reshape(EROWS, 128)
    dst = edge_index[1].reshape(EROWS, 128)
    xbits = lax.bitcast_convert_type(
        x[:, 0].astype(jnp.bfloat16), jnp.uint16).astype(jnp.uint32) << 16
    comb = lax.bitcast_convert_type(
        xbits | node_labels.astype(jnp.uint32), jnp.int32)
    comb = jnp.pad(comb, (0, NPAD - N)).reshape(NROWS, 128)

    p0, p1 = _sc_call(comb, src, dst, Param_W, Param_b)

    out = pl.pallas_call(
        _combine_body,
        out_shape=jax.ShapeDtypeStruct((NPAD // 128, 128), jnp.float32),
    )(p0.reshape(NPAD // 128, 128), p1.reshape(NPAD // 128, 128))
    return out.reshape(NPAD)[:N].reshape(N, 1)


# speedup vs baseline: 1.0169x; 1.0169x over previous
import jax
import jax.numpy as jnp
from jax import lax
from jax.experimental import pallas as pl
from jax.experimental.pallas import tpu as pltpu
from jax.experimental.pallas import tpu_sc as plsc

N = 100000
E = 6400000
L = 50
NCORES = 2
NSUB = 16
NW = NCORES * NSUB
TS = 6272
HTS = TS // 2
NPAD = NSUB * TS
CHUNK = 2048
NCHUNKS = E // CHUNK
BASE_CH = NCHUNKS // NW
EXTRA_CH = NCHUNKS % NW


def _sc_body(comb_hbm, src_hbm, dst_hbm, w_hbm, b_hbm,
             part0_hbm, part1_hbm,
             comb_v, w_v, b_v, src_a, src_b, dst_a, dst_b, msg_a, msg_b,
             idx_a, idx_b, node_v, out_sh, sem_in, sem_sc):
    srcs = (src_a, src_b)
    dsts = (dst_a, dst_b)
    msgs = (msg_a, msg_b)
    idxs = (idx_a, idx_b)
    core = lax.axis_index("c")
    sid = lax.axis_index("s")
    wid = core * NSUB + sid

    pltpu.sync_copy(comb_hbm, comb_v)
    pltpu.sync_copy(w_hbm, w_v)
    pltpu.sync_copy(b_hbm, b_v)

    def _zero(j, _):
        node_v[pl.ds(j * 16, 16)] = jnp.zeros((16,), jnp.float32)
        return _
    lax.fori_loop(0, HTS // 16, _zero, None)
    for h in range(2):
        pltpu.sync_copy(node_v, out_sh.at[pl.ds(sid * TS + h * HTS, HTS)])
    plsc.subcore_barrier()

    nch = BASE_CH + jnp.where(wid < EXTRA_CH, 1, 0)

    pltpu.sync_copy(src_hbm.at[pl.ds(wid * CHUNK, CHUNK)], srcs[0])
    pltpu.sync_copy(dst_hbm.at[pl.ds(wid * CHUNK, CHUNK)], dsts[0])

    def _one_chunk(k, p):
        off = jnp.minimum((wid + (k + 1) * NW), NCHUNKS - 1) * CHUNK
        pltpu.async_copy(src_hbm.at[pl.ds(off, CHUNK)], srcs[1 - p], sem_in)
        pltpu.async_copy(dst_hbm.at[pl.ds(off, CHUNK)], dsts[1 - p], sem_in)

        def _vec(j, _2):
            s = srcs[p][pl.ds(j * 16, 16)]
            d = dsts[p][pl.ds(j * 16, 16)]
            ws = plsc.load_gather(comb_v, [s])
            wd = plsc.load_gather(comb_v, [d])
            lab_s = ws & 0xFF
            lab_d = wd & 0xFF
            w = plsc.load_gather(w_v, [lab_d * L + lab_s])
            xf = plsc.bitcast(ws & jnp.int32(-65536), jnp.float32)
            msgs[p][pl.ds(j * 16, 16)] = w * xf
            idxs[p][pl.ds(j * 16, 16)] = d
            return _2
        lax.fori_loop(0, CHUNK // 16, _vec, None)

        @pl.when(k > 0)
        def _drain_prev():
            pltpu.make_async_copy(
                msgs[1 - p], out_sh.at[idxs[1 - p]], sem_sc).wait()

        pltpu.async_copy(msgs[p], out_sh.at[idxs[p]], sem_sc, add=True)
        pltpu.make_async_copy(src_hbm.at[pl.ds(0, CHUNK)], srcs[1 - p], sem_in).wait()
        pltpu.make_async_copy(dst_hbm.at[pl.ds(0, CHUNK)], dsts[1 - p], sem_in).wait()

    def _pair(i, _):
        for half in range(2):
            k = 2 * i + half

            @pl.when(k < nch)
            def _do(k=k, half=half):
                _one_chunk(k, half)
        return _
    lax.fori_loop(0, (BASE_CH + 2) // 2, _pair, None)

    @pl.when(nch % 2 == 1)
    def _drain_last0():
        pltpu.make_async_copy(msgs[0], out_sh.at[idxs[0]], sem_sc).wait()

    @pl.when(nch % 2 == 0)
    def _drain_last1():
        pltpu.make_async_copy(msgs[1], out_sh.at[idxs[1]], sem_sc).wait()

    plsc.subcore_barrier()

    for h in range(2):
        base = sid * TS + h * HTS
        pltpu.sync_copy(out_sh.at[pl.ds(base, HTS)], node_v)

        @pl.when(core == 0)
        def _bias(base=base):
            def _b(j, _):
                word = comb_v[pl.ds(base + j * 16, 16)]
                bias = plsc.load_gather(b_v, [word & 0xFF])
                node_v[pl.ds(j * 16, 16)] = node_v[pl.ds(j * 16, 16)] + bias
                return _
            lax.fori_loop(0, HTS // 16, _b, None)

        @pl.when(core == 0)
        def _wb0(base=base):
            pltpu.sync_copy(node_v, part0_hbm.at[pl.ds(base, HTS)])

        @pl.when(core == 1)
        def _wb1(base=base):
            pltpu.sync_copy(node_v, part1_hbm.at[pl.ds(base, HTS)])


_sc_call = pl.kernel(
    _sc_body,
    out_type=(jax.ShapeDtypeStruct((NPAD,), jnp.float32),
              jax.ShapeDtypeStruct((NPAD,), jnp.float32)),
    mesh=plsc.VectorSubcoreMesh(core_axis_name="c", subcore_axis_name="s"),
    compiler_params=pltpu.CompilerParams(needs_layout_passes=False),
    scratch_types=[
        pltpu.VMEM((NPAD,), jnp.int32),
        pltpu.VMEM((L * L,), jnp.float32),
        pltpu.VMEM((L,), jnp.float32),
        pltpu.VMEM((CHUNK,), jnp.int32),
        pltpu.VMEM((CHUNK,), jnp.int32),
        pltpu.VMEM((CHUNK,), jnp.int32),
        pltpu.VMEM((CHUNK,), jnp.int32),
        pltpu.VMEM((CHUNK,), jnp.float32),
        pltpu.VMEM((CHUNK,), jnp.float32),
        pltpu.VMEM((CHUNK,), jnp.int32),
        pltpu.VMEM((CHUNK,), jnp.int32),
        pltpu.VMEM((HTS,), jnp.float32),
        pltpu.VMEM_SHARED((NPAD,), jnp.float32),
        pltpu.SemaphoreType.DMA,
        pltpu.SemaphoreType.DMA,
    ],
)


def _combine_body(a_ref, b_ref, o_ref):
    o_ref[...] = a_ref[...] + b_ref[...]


def kernel(x, edge_index, node_labels, edge_props, Param_W, Param_b):
    del edge_props
    src = edge_index[0]
    dst = edge_index[1]
    xbits = lax.bitcast_convert_type(
        x[:, 0].astype(jnp.bfloat16), jnp.uint16).astype(jnp.uint32) << 16
    comb = lax.bitcast_convert_type(
        xbits | node_labels.astype(jnp.uint32), jnp.int32)
    comb = jnp.pad(comb, (0, NPAD - N))

    p0, p1 = _sc_call(comb, src, dst, Param_W, Param_b)

    out = pl.pallas_call(
        _combine_body,
        out_shape=jax.ShapeDtypeStruct((NPAD // 128, 128), jnp.float32),
    )(p0.reshape(NPAD // 128, 128), p1.reshape(NPAD // 128, 128))
    return out.reshape(NPAD)[:N].reshape(N, 1)
